# baseline (speedup 1.0000x reference)
import jax
import jax.numpy as jnp
from jax.experimental import pallas as pl
from jax.experimental.pallas import tpu as pltpu


NB = 8


def _body(z_ref, e_ref, zo_ref, idx_ref):
    for j in range(NB):
        _one(z_ref, e_ref, zo_ref, idx_ref, j)


def _one(z_ref, e_ref, zo_ref, idx_ref, j):
    A = z_ref[j, 0]
    E = e_ref[0]
    K = E.shape[0]
    HW = A.shape[1]
    s = jax.lax.dot_general(E, A, (((1,), (0,)), ((), ())),
                            preferred_element_type=jnp.float32)
    en = jnp.sum(E * E, axis=1, keepdims=True)
    zn = jnp.sum(A * A, axis=0, keepdims=True)
    d2 = (zn + en) - 2.0 * s
    m1 = jnp.min(d2, axis=0, keepdims=True)
    s0 = jnp.sqrt(jnp.maximum(m1, 0.0))
    mbits = jax.lax.bitcast_convert_type(m1, jnp.int32)
    T = m1
    for i in (1, 2, 3):
        ci = jax.lax.bitcast_convert_type(mbits + i, jnp.float32)
        si = jnp.sqrt(jnp.maximum(ci, 0.0))
        T = jnp.where(si == s0, ci, T)
    T = jnp.where(s0 == 0.0, 0.0, T)
    idx = jnp.argmin(jnp.maximum(d2, T), axis=0).astype(jnp.int32)
    kio = jax.lax.broadcasted_iota(jnp.int32, (K, HW), 0)
    oh = (kio == idx[None, :]).astype(jnp.float32)
    zq = jax.lax.dot_general(E, oh, (((0,), (0,)), ((), ())),
                             preferred_element_type=jnp.float32)
    zo_ref[j, 0] = A + (zq - A)
    idx_ref[0, j] = idx.reshape(idx_ref.shape[2], idx_ref.shape[3])


def kernel(z, e):
    N, ZD, H, W = z.shape
    L, K, C = e.shape
    HW = H * W
    zr = z.reshape(N, L, C, HW)
    zo, idx = pl.pallas_call(
        _body,
        grid=(L, N // NB),
        in_specs=[
            pl.BlockSpec((NB, 1, C, HW), lambda l, n: (n, l, 0, 0)),
            pl.BlockSpec((1, K, C), lambda l, n: (l, 0, 0)),
        ],
        out_specs=[
            pl.BlockSpec((NB, 1, C, HW), lambda l, n: (n, l, 0, 0)),
            pl.BlockSpec((1, NB, 8, HW // 8), lambda l, n: (l, n, 0, 0)),
        ],
        out_shape=[
            jax.ShapeDtypeStruct((N, L, C, HW), jnp.float32),
            jax.ShapeDtypeStruct((L, N, 8, HW // 8), jnp.int32),
        ],
        compiler_params=pltpu.CompilerParams(
            dimension_semantics=("parallel", "parallel")),
    )(zr, e)
    return zo.reshape(N, ZD, H, W), idx.reshape(L, N, H, W)


# speedup vs baseline: 1.0024x; 1.0024x over previous
import jax
import jax.numpy as jnp
from jax.experimental import pallas as pl
from jax.experimental.pallas import tpu as pltpu


NB = 4


def _body(z_ref, e_ref, zo_ref, idx_ref):
    for j in range(NB):
        _one(z_ref, e_ref, zo_ref, idx_ref, j)


def _one(z_ref, e_ref, zo_ref, idx_ref, j):
    A = z_ref[j, 0]
    E = e_ref[0]
    K = E.shape[0]
    HW = A.shape[1]
    s = jax.lax.dot_general(E, A, (((1,), (0,)), ((), ())),
                            preferred_element_type=jnp.float32)
    en = jnp.sum(E * E, axis=1, keepdims=True)
    zn = jnp.sum(A * A, axis=0, keepdims=True)
    d2 = (zn + en) - 2.0 * s
    m1 = jnp.min(d2, axis=0, keepdims=True)
    s0 = jnp.sqrt(jnp.maximum(m1, 0.0))
    mbits = jax.lax.bitcast_convert_type(m1, jnp.int32)
    T = m1
    for i in (1, 2, 3):
        ci = jax.lax.bitcast_convert_type(mbits + i, jnp.float32)
        si = jnp.sqrt(jnp.maximum(ci, 0.0))
        T = jnp.where(si == s0, ci, T)
    T = jnp.where(s0 == 0.0, 0.0, T)
    idx = jnp.argmin(jnp.maximum(d2, T), axis=0).astype(jnp.int32)
    kio = jax.lax.broadcasted_iota(jnp.int32, (K, HW), 0)
    oh = (kio == idx[None, :]).astype(jnp.float32)
    zq = jax.lax.dot_general(E, oh, (((0,), (0,)), ((), ())),
                             preferred_element_type=jnp.float32)
    zo_ref[j, 0] = A + (zq - A)
    idx_ref[0, j] = idx.reshape(idx_ref.shape[2], idx_ref.shape[3])


def kernel(z, e):
    N, ZD, H, W = z.shape
    L, K, C = e.shape
    HW = H * W
    zr = z.reshape(N, L, C, HW)
    zo, idx = pl.pallas_call(
        _body,
        grid=(L, N // NB),
        in_specs=[
            pl.BlockSpec((NB, 1, C, HW), lambda l, n: (n, l, 0, 0)),
            pl.BlockSpec((1, K, C), lambda l, n: (l, 0, 0)),
        ],
        out_specs=[
            pl.BlockSpec((NB, 1, C, HW), lambda l, n: (n, l, 0, 0)),
            pl.BlockSpec((1, NB, 8, HW // 8), lambda l, n: (l, n, 0, 0)),
        ],
        out_shape=[
            jax.ShapeDtypeStruct((N, L, C, HW), jnp.float32),
            jax.ShapeDtypeStruct((L, N, 8, HW // 8), jnp.int32),
        ],
        compiler_params=pltpu.CompilerParams(
            dimension_semantics=("parallel", "parallel")),
    )(zr, e)
    return zo.reshape(N, ZD, H, W), idx.reshape(L, N, H, W)


# speedup vs baseline: 1.0937x; 1.0911x over previous
import jax
import jax.numpy as jnp
from jax.experimental import pallas as pl
from jax.experimental.pallas import tpu as pltpu


NB = 4


def _body(z_ref, e_ref, zo_ref, idx_ref):
    for j in range(NB):
        _one(z_ref, e_ref, zo_ref, idx_ref, j)


def _one(z_ref, e_ref, zo_ref, idx_ref, j):
    A = z_ref[j, 0]
    E = e_ref[0]
    K = E.shape[0]
    HW = A.shape[1]
    s = jax.lax.dot_general(E, A, (((1,), (0,)), ((), ())),
                            preferred_element_type=jnp.float32)
    en = jnp.sum(E * E, axis=1, keepdims=True)
    zn = jnp.sum(A * A, axis=0, keepdims=True)
    d2 = (zn + en) - 2.0 * s
    m1 = jnp.min(d2, axis=0, keepdims=True)
    s0 = jnp.sqrt(jnp.maximum(m1, 0.0))
    mbits = jax.lax.bitcast_convert_type(m1, jnp.int32)
    T = m1
    for i in (1, 2, 3):
        ci = jax.lax.bitcast_convert_type(mbits + i, jnp.float32)
        si = jnp.sqrt(jnp.maximum(ci, 0.0))
        T = jnp.where(si == s0, ci, T)
    T = jnp.where(s0 == 0.0, 0.0, T)
    G = K // 16
    mask = jnp.where(d2 <= T, 1.0, 0.0)
    g_io = jax.lax.broadcasted_iota(jnp.int32, (G, K), 0)
    k_io = jax.lax.broadcasted_iota(jnp.int32, (G, K), 1)
    jj = k_io - 16 * g_io
    w = jnp.where((jj >= 0) & (jj < 16),
                  jax.lax.bitcast_convert_type((127 - jj) << 23, jnp.float32),
                  0.0)
    R = jax.lax.dot_general(w, mask, (((1,), (0,)), ((), ())),
                            preferred_element_type=jnp.float32)
    g_io2 = jax.lax.broadcasted_iota(jnp.int32, (G, R.shape[1]), 0)
    gsel = jnp.min(jnp.where(R > 0.0, g_io2, G), axis=0)
    v = jnp.max(jnp.where(g_io2 == gsel[None, :], R, 0.0), axis=0)
    ebits = (jax.lax.bitcast_convert_type(v, jnp.int32) >> 23) - 127
    idx = 16 * gsel - ebits
    kio = jax.lax.broadcasted_iota(jnp.int32, (K, HW), 0)
    oh = (kio == idx[None, :]).astype(jnp.float32)
    zq = jax.lax.dot_general(E, oh, (((0,), (0,)), ((), ())),
                             preferred_element_type=jnp.float32)
    zo_ref[j, 0] = A + (zq - A)
    idx_ref[0, j] = idx.reshape(idx_ref.shape[2], idx_ref.shape[3])


def kernel(z, e):
    N, ZD, H, W = z.shape
    L, K, C = e.shape
    HW = H * W
    zr = z.reshape(N, L, C, HW)
    zo, idx = pl.pallas_call(
        _body,
        grid=(L, N // NB),
        in_specs=[
            pl.BlockSpec((NB, 1, C, HW), lambda l, n: (n, l, 0, 0)),
            pl.BlockSpec((1, K, C), lambda l, n: (l, 0, 0)),
        ],
        out_specs=[
            pl.BlockSpec((NB, 1, C, HW), lambda l, n: (n, l, 0, 0)),
            pl.BlockSpec((1, NB, 8, HW // 8), lambda l, n: (l, n, 0, 0)),
        ],
        out_shape=[
            jax.ShapeDtypeStruct((N, L, C, HW), jnp.float32),
            jax.ShapeDtypeStruct((L, N, 8, HW // 8), jnp.int32),
        ],
        compiler_params=pltpu.CompilerParams(
            dimension_semantics=("parallel", "parallel")),
    )(zr, e)
    return zo.reshape(N, ZD, H, W), idx.reshape(L, N, H, W)


# speedup vs baseline: 1.0969x; 1.0030x over previous
import jax
import jax.numpy as jnp
from jax.experimental import pallas as pl
from jax.experimental.pallas import tpu as pltpu


NB = 8


def _body(z_ref, e_ref, zo_ref, idx_ref):
    for j in range(NB):
        _one(z_ref, e_ref, zo_ref, idx_ref, j)


def _one(z_ref, e_ref, zo_ref, idx_ref, j):
    A = z_ref[j, 0]
    E = e_ref[0]
    K = E.shape[0]
    HW = A.shape[1]
    s = jax.lax.dot_general(E, A, (((1,), (0,)), ((), ())),
                            preferred_element_type=jnp.float32)
    en = jnp.sum(E * E, axis=1, keepdims=True)
    zn = jnp.sum(A * A, axis=0, keepdims=True)
    d2 = (zn + en) - 2.0 * s
    m1 = jnp.min(d2, axis=0, keepdims=True)
    s0 = jnp.sqrt(jnp.maximum(m1, 0.0))
    mbits = jax.lax.bitcast_convert_type(m1, jnp.int32)
    T = m1
    for i in (1, 2, 3):
        ci = jax.lax.bitcast_convert_type(mbits + i, jnp.float32)
        si = jnp.sqrt(jnp.maximum(ci, 0.0))
        T = jnp.where(si == s0, ci, T)
    T = jnp.where(s0 == 0.0, 0.0, T)
    G = K // 16
    mask = jnp.where(d2 <= T, 1.0, 0.0)
    g_io = jax.lax.broadcasted_iota(jnp.int32, (G, K), 0)
    k_io = jax.lax.broadcasted_iota(jnp.int32, (G, K), 1)
    jj = k_io - 16 * g_io
    w = jnp.where((jj >= 0) & (jj < 16),
                  jax.lax.bitcast_convert_type((127 - jj) << 23, jnp.float32),
                  0.0)
    R = jax.lax.dot_general(w, mask, (((1,), (0,)), ((), ())),
                            preferred_element_type=jnp.float32)
    g_io2 = jax.lax.broadcasted_iota(jnp.int32, (G, R.shape[1]), 0)
    gsel = jnp.min(jnp.where(R > 0.0, g_io2, G), axis=0)
    v = jnp.max(jnp.where(g_io2 == gsel[None, :], R, 0.0), axis=0)
    ebits = (jax.lax.bitcast_convert_type(v, jnp.int32) >> 23) - 127
    idx = 16 * gsel - ebits
    kio = jax.lax.broadcasted_iota(jnp.int32, (K, HW), 0)
    oh = (kio == idx[None, :]).astype(jnp.float32)
    zq = jax.lax.dot_general(E, oh, (((0,), (0,)), ((), ())),
                             preferred_element_type=jnp.float32)
    zo_ref[j, 0] = A + (zq - A)
    idx_ref[0, j] = idx.reshape(idx_ref.shape[2], idx_ref.shape[3])


def kernel(z, e):
    N, ZD, H, W = z.shape
    L, K, C = e.shape
    HW = H * W
    zr = z.reshape(N, L, C, HW)
    zo, idx = pl.pallas_call(
        _body,
        grid=(L, N // NB),
        in_specs=[
            pl.BlockSpec((NB, 1, C, HW), lambda l, n: (n, l, 0, 0)),
            pl.BlockSpec((1, K, C), lambda l, n: (l, 0, 0)),
        ],
        out_specs=[
            pl.BlockSpec((NB, 1, C, HW), lambda l, n: (n, l, 0, 0)),
            pl.BlockSpec((1, NB, 8, HW // 8), lambda l, n: (l, n, 0, 0)),
        ],
        out_shape=[
            jax.ShapeDtypeStruct((N, L, C, HW), jnp.float32),
            jax.ShapeDtypeStruct((L, N, 8, HW // 8), jnp.int32),
        ],
        compiler_params=pltpu.CompilerParams(
            dimension_semantics=("parallel", "parallel")),
    )(zr, e)
    return zo.reshape(N, ZD, H, W), idx.reshape(L, N, H, W)
